# baseline (device time: 41157 ns/iter reference)
import jax
import jax.numpy as jnp
from jax import lax
from jax.experimental import pallas as pl
from jax.experimental.pallas import tpu as pltpu


def kernel(x, Win0, Wout0, Win1, Wout1, Win2, Wout2):
    m, d_loc = x.shape
    _, h_loc = Win0.shape
    bf16 = jnp.bfloat16

    def body(x_ref, win0_ref, wout0_ref, win1_ref, wout1_ref, win2_ref,
             wout2_ref, out_ref, win_buf, wout_buf, hsend, hrecv, gsend,
             grecv, send_sems, recv_sems, win_sem, wout_sem):
        my_x = lax.axis_index("x")
        my_y = lax.axis_index("y")
        x_partner = (1 - my_x, my_y)
        y_partner = (my_x, 1 - my_y)

        wins = (win0_ref, win1_ref, win2_ref)
        wouts = (wout0_ref, wout1_ref, wout2_ref)

        def copy_win(l):
            return pltpu.make_async_copy(wins[l], win_buf, win_sem)

        def copy_wout(l):
            return pltpu.make_async_copy(wouts[l], wout_buf, wout_sem)

        copy_win(0).start()
        copy_wout(0).start()

        barrier = pltpu.get_barrier_semaphore()
        for nbr in (x_partner, y_partner):
            pl.semaphore_signal(barrier, inc=1, device_id=nbr,
                                device_id_type=pl.DeviceIdType.MESH)
        pl.semaphore_wait(barrier, 2)

        h_rdmas = []
        g_rdmas = []
        acc = x_ref[...].astype(bf16)
        for l in range(3):
            copy_win(l).wait()
            hp = jnp.dot(acc, win_buf[...].astype(bf16),
                         preferred_element_type=jnp.float32)
            if l < 2:
                copy_win(l + 1).start()
            hsend[l] = hp.astype(bf16)
            rdma = pltpu.make_async_remote_copy(
                src_ref=hsend.at[l], dst_ref=hrecv.at[l],
                send_sem=send_sems.at[2 * l], recv_sem=recv_sems.at[2 * l],
                device_id=y_partner, device_id_type=pl.DeviceIdType.MESH)
            rdma.start()
            h_rdmas.append(rdma)
            rdma.wait_recv()
            h = jnp.maximum(hp + hrecv[l].astype(jnp.float32), 0.0)

            copy_wout(l).wait()
            gp = jnp.dot(h.astype(bf16), wout_buf[...].astype(bf16),
                         preferred_element_type=jnp.float32)
            if l < 2:
                copy_wout(l + 1).start()
            gsend[l] = gp.astype(bf16)
            rdma = pltpu.make_async_remote_copy(
                src_ref=gsend.at[l], dst_ref=grecv.at[l],
                send_sem=send_sems.at[2 * l + 1],
                recv_sem=recv_sems.at[2 * l + 1],
                device_id=x_partner, device_id_type=pl.DeviceIdType.MESH)
            rdma.start()
            g_rdmas.append(rdma)
            rdma.wait_recv()
            accf = gp + grecv[l].astype(jnp.float32)
            if l == 2:
                out_ref[...] = accf
            else:
                acc = accf.astype(bf16)

        for rdma in h_rdmas + g_rdmas:
            rdma.wait_send()

    return pl.pallas_call(
        body,
        out_shape=jax.ShapeDtypeStruct((m, d_loc), jnp.float32),
        in_specs=[pl.BlockSpec(memory_space=pltpu.VMEM)]
        + [pl.BlockSpec(memory_space=pl.ANY)] * 6,
        out_specs=pl.BlockSpec(memory_space=pltpu.VMEM),
        scratch_shapes=[
            pltpu.VMEM((d_loc, h_loc), jnp.float32),
            pltpu.VMEM((h_loc, d_loc), jnp.float32),
            pltpu.VMEM((3, m, h_loc), bf16),
            pltpu.VMEM((3, m, h_loc), bf16),
            pltpu.VMEM((3, m, d_loc), bf16),
            pltpu.VMEM((3, m, d_loc), bf16),
            pltpu.SemaphoreType.DMA((6,)),
            pltpu.SemaphoreType.DMA((6,)),
            pltpu.SemaphoreType.DMA,
            pltpu.SemaphoreType.DMA,
        ],
        compiler_params=pltpu.CompilerParams(collective_id=0),
    )(x, Win0, Wout0, Win1, Wout1, Win2, Wout2)


# device time: 40784 ns/iter; 1.0091x vs baseline; 1.0091x over previous
import jax
import jax.numpy as jnp
from jax import lax
from jax.experimental import pallas as pl
from jax.experimental.pallas import tpu as pltpu


def kernel(x, Win0, Wout0, Win1, Wout1, Win2, Wout2):
    m, d_loc = x.shape
    _, h_loc = Win0.shape
    bf16 = jnp.bfloat16

    def body(x_ref, win0_ref, wout0_ref, win1_ref, wout1_ref, win2_ref,
             wout2_ref, out_ref, win_buf, wout_buf, hsend, hrecv, gsend,
             grecv, send_sems, recv_sems, win_sem, wout_sem):
        my_x = lax.axis_index("x")
        my_y = lax.axis_index("y")
        x_partner = (1 - my_x, my_y)
        y_partner = (my_x, 1 - my_y)

        wins = (win0_ref, win1_ref, win2_ref)
        wouts = (wout0_ref, wout1_ref, wout2_ref)

        def copy_win(l):
            return pltpu.make_async_copy(wins[l], win_buf, win_sem)

        def copy_wout(l):
            return pltpu.make_async_copy(wouts[l], wout_buf, wout_sem)

        copy_win(0).start()
        copy_wout(0).start()

        barrier = pltpu.get_barrier_semaphore()
        for nbr in (x_partner, y_partner):
            pl.semaphore_signal(barrier, inc=1, device_id=nbr,
                                device_id_type=pl.DeviceIdType.MESH)
        pl.semaphore_wait(barrier, 2)

        h_rdmas = []
        g_rdmas = []
        acc = x_ref[...].astype(bf16)
        copy_win(0).wait()
        win_c = win_buf[...].astype(bf16)
        for l in range(3):
            hp = jnp.dot(acc, win_c, preferred_element_type=jnp.float32)
            if l < 2:
                copy_win(l + 1).start()
            hsend[l] = hp.astype(bf16)
            rdma = pltpu.make_async_remote_copy(
                src_ref=hsend.at[l], dst_ref=hrecv.at[l],
                send_sem=send_sems.at[2 * l], recv_sem=recv_sems.at[2 * l],
                device_id=y_partner, device_id_type=pl.DeviceIdType.MESH)
            rdma.start()
            h_rdmas.append(rdma)
            copy_wout(l).wait()
            wout_c = wout_buf[...].astype(bf16)
            rdma.wait_recv()
            h = jnp.maximum(hp + hrecv[l].astype(jnp.float32), 0.0)

            gp = jnp.dot(h.astype(bf16), wout_c,
                         preferred_element_type=jnp.float32)
            if l < 2:
                copy_wout(l + 1).start()
            gsend[l] = gp.astype(bf16)
            rdma = pltpu.make_async_remote_copy(
                src_ref=gsend.at[l], dst_ref=grecv.at[l],
                send_sem=send_sems.at[2 * l + 1],
                recv_sem=recv_sems.at[2 * l + 1],
                device_id=x_partner, device_id_type=pl.DeviceIdType.MESH)
            rdma.start()
            g_rdmas.append(rdma)
            if l < 2:
                copy_win(l + 1).wait()
                win_c = win_buf[...].astype(bf16)
            rdma.wait_recv()
            accf = gp + grecv[l].astype(jnp.float32)
            if l == 2:
                out_ref[...] = accf
            else:
                acc = accf.astype(bf16)

        for rdma in h_rdmas + g_rdmas:
            rdma.wait_send()

    return pl.pallas_call(
        body,
        out_shape=jax.ShapeDtypeStruct((m, d_loc), jnp.float32),
        in_specs=[pl.BlockSpec(memory_space=pltpu.VMEM)]
        + [pl.BlockSpec(memory_space=pl.ANY)] * 6,
        out_specs=pl.BlockSpec(memory_space=pltpu.VMEM),
        scratch_shapes=[
            pltpu.VMEM((d_loc, h_loc), jnp.float32),
            pltpu.VMEM((h_loc, d_loc), jnp.float32),
            pltpu.VMEM((3, m, h_loc), bf16),
            pltpu.VMEM((3, m, h_loc), bf16),
            pltpu.VMEM((3, m, d_loc), bf16),
            pltpu.VMEM((3, m, d_loc), bf16),
            pltpu.SemaphoreType.DMA((6,)),
            pltpu.SemaphoreType.DMA((6,)),
            pltpu.SemaphoreType.DMA,
            pltpu.SemaphoreType.DMA,
        ],
        compiler_params=pltpu.CompilerParams(collective_id=0),
    )(x, Win0, Wout0, Win1, Wout1, Win2, Wout2)
